# baseline (speedup 1.0000x reference)
import functools

import jax
import jax.numpy as jnp
from jax.experimental import pallas as pl
from jax.experimental.pallas import tpu as pltpu

N = 4096
K = N // 2
LOGN = 12
ROWS = 128


def _row_step(xp, j, k, logd):
    m = 1 << (logd + 3)
    xr = xp.reshape(N // (2 * m), 2, m, ROWS)
    a = xr[:, 0]
    b = xr[:, 1]
    mx = jnp.maximum(a, b)
    mn = jnp.minimum(a, b)
    dm = ((j & k) == 0).reshape(N // (2 * m), 2, m, 1)[:, 0]
    sel_lo = jnp.where(dm, mx, mn)
    sel_hi = jnp.where(dm, mn, mx)
    return jnp.stack([sel_lo, sel_hi], axis=1).reshape(N, ROWS)


def _sub_step(xp, j, q, k, logd):
    dp = 1 << (logd - 9)
    xs = xp.reshape(N // 8, 8 // (2 * dp), 2, dp, ROWS)
    partner = jnp.stack([xs[:, :, 1], xs[:, :, 0]], axis=2).reshape(N, ROWS)
    lower = (q & dp) == 0
    keep_max = ((j & k) == 0) == lower
    return jnp.where(keep_max, jnp.maximum(xp, partner),
                     jnp.minimum(xp, partner))


def _sort_body(x_ref, o_ref):
    x = x_ref[...]
    parts = [jnp.transpose(x[:, h * 512:(h + 1) * 512]) for h in range(8)]
    xp = jnp.stack(parts, axis=1).reshape(N, ROWS)

    q = jax.lax.broadcasted_iota(jnp.int32, (N, 1), 0)
    j = (q % 8) * 512 + (q // 8)

    for logk in range(1, LOGN + 1):
        k = 1 << logk
        for logd in range(logk - 1, -1, -1):
            if logd >= 9:
                xp = _sub_step(xp, j, q, k, logd)
            else:
                xp = _row_step(xp, j, k, logd)

    xs = xp.reshape(N // 8, 8, ROWS)
    for h in range(4):
        o_ref[:, h * 512:(h + 1) * 512] = jnp.transpose(xs[:, h, :])


@jax.jit
def kernel(x):
    b, t, n = x.shape
    rows = b * t
    flat = x.reshape(rows, n)
    out = pl.pallas_call(
        _sort_body,
        grid=(rows // ROWS,),
        in_specs=[pl.BlockSpec((ROWS, N), lambda i: (i, 0))],
        out_specs=pl.BlockSpec((ROWS, K), lambda i: (i, 0)),
        out_shape=jax.ShapeDtypeStruct((rows, K), jnp.float32),
        compiler_params=pltpu.CompilerParams(
            dimension_semantics=("arbitrary",),
        ),
    )(flat)
    return out.reshape(b, t, K)


# speedup vs baseline: 1.0218x; 1.0218x over previous
import functools

import jax
import jax.numpy as jnp
from jax.experimental import pallas as pl
from jax.experimental.pallas import tpu as pltpu

N = 4096
K = N // 2
LOGN = 12
ROWS = 128


def _row_step(xp, j, k, logd):
    n = xp.shape[0]
    m = 1 << (logd + 3)
    xr = xp.reshape(n // (2 * m), 2, m, ROWS)
    a = xr[:, 0]
    b = xr[:, 1]
    mx = jnp.maximum(a, b)
    mn = jnp.minimum(a, b)
    dm = ((j & k) == 0).reshape(n // (2 * m), 2, m, 1)[:, 0]
    sel_lo = jnp.where(dm, mx, mn)
    sel_hi = jnp.where(dm, mn, mx)
    return jnp.stack([sel_lo, sel_hi], axis=1).reshape(n, ROWS)


def _plane_pair_step(ys, k, logd):
    hp = 1 << (logd - 9)
    out = list(ys)
    for h in range(8):
        if h & hp:
            continue
        h2 = h | hp
        desc = ((h << 9) & k) == 0
        mx = jnp.maximum(ys[h], ys[h2])
        mn = jnp.minimum(ys[h], ys[h2])
        out[h], out[h2] = (mx, mn) if desc else (mn, mx)
    return out


def _plane_row_step(y, desc, logd):
    m = 1 << logd
    yr = y.reshape(512 // (2 * m), 2, m, ROWS)
    mx = jnp.maximum(yr[:, 0], yr[:, 1])
    mn = jnp.minimum(yr[:, 0], yr[:, 1])
    pair = (mx, mn) if desc else (mn, mx)
    return jnp.stack(pair, axis=1).reshape(512, ROWS)


def _sort_body(x_ref, o_ref):
    x = x_ref[...]
    parts = [jnp.transpose(x[:, h * 512:(h + 1) * 512]) for h in range(8)]
    xp = jnp.stack(parts, axis=1).reshape(N, ROWS)

    q = jax.lax.broadcasted_iota(jnp.int32, (N, 1), 0)
    j = (q % 8) * 512 + (q // 8)

    for logk in range(1, 10):
        k = 1 << logk
        for logd in range(logk - 1, -1, -1):
            xp = _row_step(xp, j, k, logd)

    for logk in (10, 11):
        k = 1 << logk
        ys = [xp.reshape(N // 8, 8, ROWS)[:, h, :] for h in range(8)]
        for logd in range(logk - 1, 8, -1):
            ys = _plane_pair_step(ys, k, logd)
        for logd in range(8, 2, -1):
            ys = [_plane_row_step(y, ((h << 9) & k) == 0, logd)
                  for h, y in enumerate(ys)]
        xp = jnp.stack(ys, axis=1).reshape(N, ROWS)
        for logd in (2, 1, 0):
            xp = _row_step(xp, j, k, logd)

    k = 1 << LOGN
    ys = [xp.reshape(N // 8, 8, ROWS)[:, h, :] for h in range(8)]
    for logd in (11, 10, 9):
        ys = _plane_pair_step(ys, k, logd)
    ys = ys[:4]
    for logd in range(8, 2, -1):
        ys = [_plane_row_step(y, True, logd) for y in ys]

    semi = [ys[s & 3].reshape(2, 256, ROWS)[s >> 2] for s in range(8)]
    z = jnp.stack(semi, axis=1).reshape(K, ROWS)
    jz = jnp.zeros((K, 1), jnp.int32)
    for logd in (2, 1, 0):
        z = _row_step(z, jz, k, logd)

    zs = z.reshape(K // 8, 8, ROWS)
    for s in range(8):
        base = (s & 1) * 512 + ((s >> 1) & 1) * 1024 + (s >> 2) * 256
        o_ref[:, base:base + 256] = jnp.transpose(zs[:, s, :])


@jax.jit
def kernel(x):
    b, t, n = x.shape
    rows = b * t
    flat = x.reshape(rows, n)
    out = pl.pallas_call(
        _sort_body,
        grid=(rows // ROWS,),
        in_specs=[pl.BlockSpec((ROWS, N), lambda i: (i, 0))],
        out_specs=pl.BlockSpec((ROWS, K), lambda i: (i, 0)),
        out_shape=jax.ShapeDtypeStruct((rows, K), jnp.float32),
        compiler_params=pltpu.CompilerParams(
            dimension_semantics=("arbitrary",),
        ),
    )(flat)
    return out.reshape(b, t, K)


# speedup vs baseline: 1.1180x; 1.0942x over previous
import functools

import jax
import jax.numpy as jnp
from jax.experimental import pallas as pl
from jax.experimental.pallas import tpu as pltpu

N = 4096
K = N // 2
LOGN = 12
ROWS = 128


def _row_step(xp, j, k, logd):
    n = xp.shape[0]
    m = 1 << (logd + 3)
    xr = xp.reshape(n // (2 * m), 2, m, ROWS)
    a = xr[:, 0]
    b = xr[:, 1]
    mx = jnp.maximum(a, b)
    mn = jnp.minimum(a, b)
    dm = ((j & k) == 0).reshape(n // (2 * m), 2, m, 1)[:, 0]
    sel_lo = jnp.where(dm, mx, mn)
    sel_hi = jnp.where(dm, mn, mx)
    return jnp.stack([sel_lo, sel_hi], axis=1).reshape(n, ROWS)


def _sub_step(xp, j, q, k, logd):
    dp = 1 << (logd - 9)
    pu = jnp.roll(xp, dp, axis=0)
    pd = jnp.roll(xp, -dp, axis=0)
    lower = (q & dp) == 0
    partner = jnp.where(lower, pd, pu)
    keep_max = ((j & k) == 0) == lower
    return jnp.where(keep_max, jnp.maximum(xp, partner),
                     jnp.minimum(xp, partner))


def _plane_pair_step(ys, k, logd):
    hp = 1 << (logd - 9)
    out = list(ys)
    for h in range(8):
        if h & hp:
            continue
        h2 = h | hp
        desc = ((h << 9) & k) == 0
        mx = jnp.maximum(ys[h], ys[h2])
        mn = jnp.minimum(ys[h], ys[h2])
        out[h], out[h2] = (mx, mn) if desc else (mn, mx)
    return out


def _plane_row_step(y, desc, logd):
    m = 1 << logd
    yr = y.reshape(512 // (2 * m), 2, m, ROWS)
    mx = jnp.maximum(yr[:, 0], yr[:, 1])
    mn = jnp.minimum(yr[:, 0], yr[:, 1])
    pair = (mx, mn) if desc else (mn, mx)
    return jnp.stack(pair, axis=1).reshape(512, ROWS)


def _sort_body(x_ref, o_ref):
    x = x_ref[...]
    parts = [jnp.transpose(x[:, h * 512:(h + 1) * 512]) for h in range(8)]
    xp = jnp.stack(parts, axis=1).reshape(N, ROWS)

    q = jax.lax.broadcasted_iota(jnp.int32, (N, 1), 0)
    j = (q % 8) * 512 + (q // 8)

    for logk in range(1, 10):
        k = 1 << logk
        for logd in range(logk - 1, -1, -1):
            xp = _row_step(xp, j, k, logd)

    k = 1 << 10
    xp = _sub_step(xp, j, q, k, 9)
    for logd in range(8, -1, -1):
        xp = _row_step(xp, j, k, logd)

    k = 1 << 11
    ys = [xp.reshape(N // 8, 8, ROWS)[:, h, :] for h in range(8)]
    for logd in (10, 9):
        ys = _plane_pair_step(ys, k, logd)
    xp = jnp.stack(ys, axis=1).reshape(N, ROWS)
    for logd in range(8, -1, -1):
        xp = _row_step(xp, j, k, logd)

    k = 1 << LOGN
    ys = [xp.reshape(N // 8, 8, ROWS)[:, h, :] for h in range(8)]
    for logd in (11, 10, 9):
        ys = _plane_pair_step(ys, k, logd)
    ys = [_plane_row_step(y, True, 8) for y in ys[:4]]

    semi = [ys[s & 3].reshape(2, 256, ROWS)[s >> 2] for s in range(8)]
    z = jnp.stack(semi, axis=1).reshape(K, ROWS)
    jz = jnp.zeros((K, 1), jnp.int32)
    for logd in range(7, -1, -1):
        z = _row_step(z, jz, k, logd)

    zs = z.reshape(K // 8, 8, ROWS)
    for s in range(8):
        base = (s & 1) * 512 + ((s >> 1) & 1) * 1024 + (s >> 2) * 256
        o_ref[:, base:base + 256] = jnp.transpose(zs[:, s, :])


@jax.jit
def kernel(x):
    b, t, n = x.shape
    rows = b * t
    flat = x.reshape(rows, n)
    out = pl.pallas_call(
        _sort_body,
        grid=(rows // ROWS,),
        in_specs=[pl.BlockSpec((ROWS, N), lambda i: (i, 0))],
        out_specs=pl.BlockSpec((ROWS, K), lambda i: (i, 0)),
        out_shape=jax.ShapeDtypeStruct((rows, K), jnp.float32),
        compiler_params=pltpu.CompilerParams(
            dimension_semantics=("arbitrary",),
        ),
    )(flat)
    return out.reshape(b, t, K)


# speedup vs baseline: 1.9220x; 1.7191x over previous
import functools

import jax
import jax.numpy as jnp
from jax.experimental import pallas as pl
from jax.experimental.pallas import tpu as pltpu

N = 4096
K = N // 2
LOGN = 12
ROWS = 128


def _row_step(xp, logd):
    n = xp.shape[0]
    m = 1 << (logd + 3)
    xr = xp.reshape(n // (2 * m), 2, m, ROWS)
    mx = jnp.maximum(xr[:, 0], xr[:, 1])
    mn = jnp.minimum(xr[:, 0], xr[:, 1])
    return jnp.stack([mx, mn], axis=1).reshape(n, ROWS)


def _sub_step(xp, q, logd):
    dp = 1 << (logd - 9)
    pu = jnp.roll(xp, dp, axis=0)
    pd = jnp.roll(xp, -dp, axis=0)
    lower = (q & dp) == 0
    return jnp.where(lower, jnp.maximum(xp, pd), jnp.minimum(xp, pu))


def _plane_pair_step(ys, logd):
    hp = 1 << (logd - 9)
    out = list(ys)
    for h in range(8):
        if h & hp:
            continue
        h2 = h | hp
        out[h] = jnp.maximum(ys[h], ys[h2])
        out[h2] = jnp.minimum(ys[h], ys[h2])
    return out


def _sort_body(x_ref, o_ref):
    x = x_ref[...]
    parts = [jnp.transpose(x[:, h * 512:(h + 1) * 512]) for h in range(8)]
    xp = jnp.stack(parts, axis=1).reshape(N, ROWS)

    q = jax.lax.broadcasted_iota(jnp.int32, (N, 1), 0)
    j = (q % 8) * 512 + (q // 8)

    def sgn(k):
        return jnp.where((j & k) == 0, jnp.float32(1), jnp.float32(-1))

    xp = xp * sgn(2)
    for logk in range(1, 10):
        for logd in range(logk - 1, -1, -1):
            xp = _row_step(xp, logd)
        xp = xp * (sgn(1 << logk) * sgn(2 << logk))

    xp = _sub_step(xp, q, 9)
    for logd in range(8, -1, -1):
        xp = _row_step(xp, logd)
    xp = xp * (sgn(1 << 10) * sgn(1 << 11))

    ys = [xp.reshape(N // 8, 8, ROWS)[:, h, :] for h in range(8)]
    for logd in (10, 9):
        ys = _plane_pair_step(ys, logd)
    xp = jnp.stack(ys, axis=1).reshape(N, ROWS)
    for logd in range(8, -1, -1):
        xp = _row_step(xp, logd)
    xp = xp * sgn(1 << 11)

    ys = [xp.reshape(N // 8, 8, ROWS)[:, h, :] for h in range(8)]
    for logd in (11, 10, 9):
        ys = _plane_pair_step(ys, logd)
    ys = [_row_step(y, 5) for y in ys[:4]]

    semi = [ys[s & 3].reshape(2, 256, ROWS)[s >> 2] for s in range(8)]
    z = jnp.stack(semi, axis=1).reshape(K, ROWS)
    for logd in range(7, -1, -1):
        z = _row_step(z, logd)

    zs = z.reshape(K // 8, 8, ROWS)
    for s in range(8):
        base = (s & 1) * 512 + ((s >> 1) & 1) * 1024 + (s >> 2) * 256
        o_ref[:, base:base + 256] = jnp.transpose(zs[:, s, :])


@jax.jit
def kernel(x):
    b, t, n = x.shape
    rows = b * t
    flat = x.reshape(rows, n)
    out = pl.pallas_call(
        _sort_body,
        grid=(rows // ROWS,),
        in_specs=[pl.BlockSpec((ROWS, N), lambda i: (i, 0))],
        out_specs=pl.BlockSpec((ROWS, K), lambda i: (i, 0)),
        out_shape=jax.ShapeDtypeStruct((rows, K), jnp.float32),
        compiler_params=pltpu.CompilerParams(
            dimension_semantics=("arbitrary",),
        ),
    )(flat)
    return out.reshape(b, t, K)
